# Initial kernel scaffold; baseline (speedup 1.0000x reference)
import jax
import jax.numpy as jnp
from jax.experimental import pallas as pl


def kernel(adj_row, adj_col, adj_val, ent_emb, basis_weights, basis_coeff):
    raise NotImplementedError("write your pallas kernel here")


# speedup vs baseline: 1.0983x; 1.0983x over previous
import functools

import jax
import jax.numpy as jnp
from jax import lax
from jax.experimental import pallas as pl
from jax.experimental.pallas import tpu as pltpu

N = 50000
D = 128
NB = 2
ROW_BLK = 400


def _transform_body(cvec_ref, w_ref, acc_ref, out_ref, *, last):
    w = cvec_ref[0, 0] * w_ref[0] + cvec_ref[0, 1] * w_ref[1]
    y = jnp.dot(acc_ref[...], w.T, preferred_element_type=jnp.float32)
    y = jnp.maximum(y, 0.0)
    if last:
        nrm = jnp.sqrt(jnp.sum(y * y, axis=1, keepdims=True))
        y = y / jnp.maximum(nrm, 1e-12)
    out_ref[...] = y


def _transform(acc, weights_l, coeff_l, last):
    cvec = coeff_l.reshape(1, NB)
    grid = N // ROW_BLK
    return pl.pallas_call(
        functools.partial(_transform_body, last=last),
        grid=(grid,),
        in_specs=[
            pl.BlockSpec((1, NB), lambda i: (0, 0)),
            pl.BlockSpec((NB, D, D), lambda i: (0, 0, 0)),
            pl.BlockSpec((ROW_BLK, D), lambda i: (i, 0)),
        ],
        out_specs=pl.BlockSpec((ROW_BLK, D), lambda i: (i, 0)),
        out_shape=jax.ShapeDtypeStruct((N, D), jnp.float32),
    )(cvec, weights_l, acc)


def kernel(adj_row, adj_col, adj_val, ent_emb, basis_weights, basis_coeff):
    rows = adj_row.reshape(-1).astype(jnp.int32)
    cols = adj_col.reshape(-1).astype(jnp.int32)
    vals = adj_val.reshape(-1)
    emb = ent_emb
    L = basis_weights.shape[0]
    for l in range(L):
        msgs = vals[:, None] * emb[cols]
        acc = jax.ops.segment_sum(msgs, rows, num_segments=N)
        emb = _transform(acc, basis_weights[l], basis_coeff[l, l], last=(l == L - 1))
    return emb
